# baseline (speedup 1.0000x reference)
import functools

import jax
import jax.numpy as jnp
from jax import lax
from jax.experimental import pallas as pl
from jax.experimental.pallas import tpu as pltpu
from jax.experimental.pallas import tpu_sc as plsc

ROWS = 64
COLS = 32768
K = 64
NVREG = COLS // 16
NGROUP = NVREG // 16
NEG = float("-inf")

_info = plsc.get_sparse_core_info()
NC, NS = _info.num_cores, _info.num_subcores
NW = NC * NS
ROWS_PER_W = ROWS // NW


def _sort_asc(v):
    return lax.sort(v, dimension=0)


_XORIDX = {}
_DMASK = {}


def _cleanup_asc(v):
    lane = lax.iota(jnp.int32, 16)
    for d in (8, 4, 2, 1):
        idx = (lane ^ d).reshape(16, 1)
        p = lax.gather(v, idx, _GDN, (1,),
                       mode=lax.GatherScatterMode.PROMISE_IN_BOUNDS)
        lo = jnp.minimum(v, p)
        hi = jnp.maximum(v, p)
        v = jnp.where((lane & d) != 0, hi, lo)
    return v


def _merge(A, b):
    b_desc = lax.rev(_sort_asc(b), dimensions=(0,))
    h0 = jnp.maximum(A[0], b_desc)
    p0 = jnp.minimum(h0, A[2])
    p2 = jnp.maximum(h0, A[2])
    q0 = jnp.minimum(p0, A[1])
    q1 = jnp.maximum(p0, A[1])
    q2 = jnp.minimum(p2, A[3])
    q3 = jnp.maximum(p2, A[3])
    return (_cleanup_asc(q0), _cleanup_asc(q1),
            _cleanup_asc(q2), _cleanup_asc(q3))


def _neg_buf():
    z = jnp.full((16,), NEG, jnp.float32)
    return (z, z, z, z)


_GDN = lax.GatherDimensionNumbers(
    offset_dims=(), collapsed_slice_dims=(0,), start_index_map=(0,))


def _bcast0(v):
    idx = jnp.zeros((16, 1), jnp.int32)
    return lax.gather(v, idx, _GDN, (1,),
                      mode=lax.GatherScatterMode.PROMISE_IN_BOUNDS)


@functools.partial(
    pl.kernel,
    out_type=jax.ShapeDtypeStruct((ROWS, K), jnp.float32),
    mesh=plsc.VectorSubcoreMesh(core_axis_name="c", subcore_axis_name="s"),
    compiler_params=pltpu.CompilerParams(needs_layout_passes=False),
    scratch_types=[
        pltpu.VMEM((COLS,), jnp.float32),
        pltpu.VMEM((COLS,), jnp.float32),
        pltpu.VMEM((NGROUP * 16,), jnp.float32),
        pltpu.VMEM((NGROUP * 16 + 16,), jnp.int32),
        pltpu.VMEM((K,), jnp.float32),
        pltpu.SemaphoreType.DMA,
        pltpu.SemaphoreType.DMA,
    ],
)
def _topk_sc(x_hbm, out_hbm, x_v0, x_v1, m_v, cand_v, res_v, sem0, sem1):
    wid = lax.axis_index("s") * NC + lax.axis_index("c")
    lane = lax.iota(jnp.int32, 16)

    row0 = wid * ROWS_PER_W
    cp0 = pltpu.async_copy(x_hbm.at[row0], x_v0, sem0)
    cp1 = pltpu.async_copy(x_hbm.at[row0 + 1], x_v1, sem1)

    def process_row(x_v, r):
        def bucket_body(g, T):
            base = g * 256
            acc = x_v[pl.ds(base, 16)]
            for j in range(1, 16):
                acc = jnp.maximum(acc, x_v[pl.ds(base + j * 16, 16)])
            m_v[pl.ds(g * 16, 16)] = acc
            t = acc
            T2 = []
            for s in range(8):
                T2.append(jnp.maximum(T[s], t))
                t = jnp.minimum(T[s], t)
            return tuple(T2)

        z = jnp.full((16,), NEG, jnp.float32)
        T = lax.fori_loop(0, NGROUP, bucket_body, (z,) * 8)

        AM = _neg_buf()
        for s in range(8):
            AM = _merge(AM, T[s])
        tau_v = _bcast0(AM[0])

        def comp_body(gq, offs):
            new_offs = []
            for u in range(4):
                g = u * (NGROUP // 4) + gq
                mg = m_v[pl.ds(g * 16, 16)]
                m = mg >= tau_v
                ids = g * 16 + lane
                plsc.store_compressed(
                    cand_v.at[pl.ds(u * 512 + offs[u], 16)], ids, mask=m)
                new_offs.append(
                    offs[u] + plsc.all_reduce_population_count(m)[0])
            return tuple(new_offs)

        zero = jnp.int32(0)
        counts = lax.fori_loop(0, NGROUP // 4, comp_body,
                               (zero, zero, zero, zero))

        mcount = jnp.maximum(jnp.maximum(counts[0], counts[1]),
                             jnp.maximum(counts[2], counts[3]))
        count_vs = [jnp.full((16,), c, jnp.int32) for c in counts]
        negv = jnp.full((16,), NEG, jnp.float32)

        def cand_body(i, Fs):
            i_v = jnp.full((16,), i, jnp.int32)
            new_Fs = []
            for u in range(4):
                cid = cand_v[pl.ds(u * 512 + i, 16)][0]
                idx = ((cid // 16) * 256 + lane * 16 + (cid % 16)) & (COLS - 1)
                b = plsc.load_gather(x_v, [idx])
                b = jnp.where(i_v < count_vs[u], b, negv)
                new_Fs.append(_merge(Fs[u], b))
            return tuple(new_Fs)

        Fs = lax.fori_loop(0, mcount, cand_body, (_neg_buf(),) * 4)

        def merge64(A, B):
            k = [jnp.maximum(A[i], lax.rev(B[3 - i], dimensions=(0,)))
                 for i in range(4)]
            p0 = jnp.minimum(k[0], k[2])
            p2 = jnp.maximum(k[0], k[2])
            p1 = jnp.minimum(k[1], k[3])
            p3 = jnp.maximum(k[1], k[3])
            q0 = jnp.minimum(p0, p1)
            q1 = jnp.maximum(p0, p1)
            q2 = jnp.minimum(p2, p3)
            q3 = jnp.maximum(p2, p3)
            return (_sort_asc(q0), _sort_asc(q1),
                    _sort_asc(q2), _sort_asc(q3))

        F = merge64(merge64(Fs[0], Fs[1]), merge64(Fs[2], Fs[3]))

        for j in range(4):
            res_v[pl.ds(j * 16, 16)] = lax.rev(F[3 - j], dimensions=(0,))
        pltpu.sync_copy(res_v, out_hbm.at[r])

    cp0.wait()
    process_row(x_v0, row0)
    cp1.wait()
    process_row(x_v1, row0 + 1)


def kernel(x):
    return _topk_sc(x).reshape(1, ROWS * K)


# speedup vs baseline: 1.1002x; 1.1002x over previous
import functools

import jax
import jax.numpy as jnp
from jax import lax
from jax.experimental import pallas as pl
from jax.experimental.pallas import tpu as pltpu
from jax.experimental.pallas import tpu_sc as plsc

ROWS = 64
COLS = 32768
K = 64
NVREG = COLS // 16
NGROUP = NVREG // 16
NEG = float("-inf")

_info = plsc.get_sparse_core_info()
NC, NS = _info.num_cores, _info.num_subcores
NW = NC * NS
ROWS_PER_W = ROWS // NW


def _sort_asc(v):
    return lax.sort(v, dimension=0)


def _merge(A, b):
    b_desc = -_sort_asc(-b)
    h0 = jnp.maximum(A[0], b_desc)
    p0 = jnp.minimum(h0, A[2])
    p2 = jnp.maximum(h0, A[2])
    q0 = jnp.minimum(p0, A[1])
    q1 = jnp.maximum(p0, A[1])
    q2 = jnp.minimum(p2, A[3])
    q3 = jnp.maximum(p2, A[3])
    return (_sort_asc(q0), _sort_asc(q1), _sort_asc(q2), _sort_asc(q3))


def _neg_buf():
    z = jnp.full((16,), NEG, jnp.float32)
    return (z, z, z, z)


_GDN = lax.GatherDimensionNumbers(
    offset_dims=(), collapsed_slice_dims=(0,), start_index_map=(0,))


def _bcast0(v):
    idx = jnp.zeros((16, 1), jnp.int32)
    return lax.gather(v, idx, _GDN, (1,),
                      mode=lax.GatherScatterMode.PROMISE_IN_BOUNDS)


@functools.partial(
    pl.kernel,
    out_type=jax.ShapeDtypeStruct((ROWS, K), jnp.float32),
    mesh=plsc.VectorSubcoreMesh(core_axis_name="c", subcore_axis_name="s"),
    compiler_params=pltpu.CompilerParams(needs_layout_passes=False),
    scratch_types=[
        pltpu.VMEM((COLS,), jnp.float32),
        pltpu.VMEM((COLS,), jnp.float32),
        pltpu.VMEM((NGROUP * 16,), jnp.float32),
        pltpu.VMEM((NGROUP * 16 + 16,), jnp.int32),
        pltpu.VMEM((K,), jnp.float32),
        pltpu.SemaphoreType.DMA,
        pltpu.SemaphoreType.DMA,
    ],
)
def _topk_sc(x_hbm, out_hbm, x_v0, x_v1, m_v, cand_v, res_v, sem0, sem1):
    wid = lax.axis_index("s") * NC + lax.axis_index("c")
    lane = lax.iota(jnp.int32, 16)

    row0 = wid * ROWS_PER_W
    cp0 = pltpu.async_copy(x_hbm.at[row0], x_v0, sem0)
    cp1 = pltpu.async_copy(x_hbm.at[row0 + 1], x_v1, sem1)

    def process_row(x_v, r):
        def bucket_body(g, T):
            base = g * 256
            acc = x_v[pl.ds(base, 16)]
            for j in range(1, 16):
                acc = jnp.maximum(acc, x_v[pl.ds(base + j * 16, 16)])
            m_v[pl.ds(g * 16, 16)] = acc
            t = acc
            T2 = []
            for s in range(8):
                T2.append(jnp.maximum(T[s], t))
                t = jnp.minimum(T[s], t)
            return tuple(T2)

        z = jnp.full((16,), NEG, jnp.float32)
        T = lax.fori_loop(0, NGROUP, bucket_body, (z,) * 8)

        AM = _neg_buf()
        for s in range(8):
            AM = _merge(AM, T[s])
        tau_v = _bcast0(AM[0])

        def comp_body(gq, offs):
            new_offs = []
            for u in range(4):
                g = u * (NGROUP // 4) + gq
                mg = m_v[pl.ds(g * 16, 16)]
                m = mg >= tau_v
                ids = g * 16 + lane
                plsc.store_compressed(
                    cand_v.at[pl.ds(u * 512 + offs[u], 16)], ids, mask=m)
                new_offs.append(
                    offs[u] + plsc.all_reduce_population_count(m)[0])
            return tuple(new_offs)

        zero = jnp.int32(0)
        counts = lax.fori_loop(0, NGROUP // 4, comp_body,
                               (zero, zero, zero, zero))

        mcount = jnp.maximum(jnp.maximum(counts[0], counts[1]),
                             jnp.maximum(counts[2], counts[3]))
        count_vs = [jnp.full((16,), c, jnp.int32) for c in counts]
        negv = jnp.full((16,), NEG, jnp.float32)

        def cand_body(i, carry):
            Fs, i_v = carry
            new_Fs = []
            for u in range(4):
                cid = cand_v[pl.ds(u * 512 + i, 16)][0]
                idx = ((cid // 16) * 256 + lane * 16 + (cid % 16)) & (COLS - 1)
                b = plsc.load_gather(x_v, [idx])
                b = jnp.where(i_v < count_vs[u], b, negv)
                new_Fs.append(_merge(Fs[u], b))
            return tuple(new_Fs), i_v + 1

        Fs, _ = lax.fori_loop(0, mcount, cand_body,
                              ((_neg_buf(),) * 4, jnp.zeros((16,), jnp.int32)))

        def merge64(A, B):
            k = [jnp.maximum(A[i], lax.rev(B[3 - i], dimensions=(0,)))
                 for i in range(4)]
            p0 = jnp.minimum(k[0], k[2])
            p2 = jnp.maximum(k[0], k[2])
            p1 = jnp.minimum(k[1], k[3])
            p3 = jnp.maximum(k[1], k[3])
            q0 = jnp.minimum(p0, p1)
            q1 = jnp.maximum(p0, p1)
            q2 = jnp.minimum(p2, p3)
            q3 = jnp.maximum(p2, p3)
            return (_sort_asc(q0), _sort_asc(q1),
                    _sort_asc(q2), _sort_asc(q3))

        F = merge64(merge64(Fs[0], Fs[1]), merge64(Fs[2], Fs[3]))

        for j in range(4):
            res_v[pl.ds(j * 16, 16)] = lax.rev(F[3 - j], dimensions=(0,))
        pltpu.sync_copy(res_v, out_hbm.at[r])

    cp0.wait()
    process_row(x_v0, row0)
    cp1.wait()
    process_row(x_v1, row0 + 1)


def kernel(x):
    return _topk_sc(x).reshape(1, ROWS * K)


# speedup vs baseline: 1.1078x; 1.0069x over previous
import functools

import jax
import jax.numpy as jnp
from jax import lax
from jax.experimental import pallas as pl
from jax.experimental.pallas import tpu as pltpu
from jax.experimental.pallas import tpu_sc as plsc

ROWS = 64
COLS = 32768
K = 64
NVREG = COLS // 16
NGROUP = NVREG // 16
NEG = float("-inf")

_info = plsc.get_sparse_core_info()
NC, NS = _info.num_cores, _info.num_subcores
NW = NC * NS
ROWS_PER_W = ROWS // NW


def _sort_asc(v):
    return lax.sort(v, dimension=0)


def _merge(A, b):
    b_desc = lax.rev(_sort_asc(b), dimensions=(0,))
    h0 = jnp.maximum(A[0], b_desc)
    p0 = jnp.minimum(h0, A[2])
    p2 = jnp.maximum(h0, A[2])
    q0 = jnp.minimum(p0, A[1])
    q1 = jnp.maximum(p0, A[1])
    q2 = jnp.minimum(p2, A[3])
    q3 = jnp.maximum(p2, A[3])
    return (_sort_asc(q0), _sort_asc(q1), _sort_asc(q2), _sort_asc(q3))


def _neg_buf():
    z = jnp.full((16,), NEG, jnp.float32)
    return (z, z, z, z)


_GDN = lax.GatherDimensionNumbers(
    offset_dims=(), collapsed_slice_dims=(0,), start_index_map=(0,))


def _bcast0(v):
    idx = jnp.zeros((16, 1), jnp.int32)
    return lax.gather(v, idx, _GDN, (1,),
                      mode=lax.GatherScatterMode.PROMISE_IN_BOUNDS)


@functools.partial(
    pl.kernel,
    out_type=jax.ShapeDtypeStruct((ROWS, K), jnp.float32),
    mesh=plsc.VectorSubcoreMesh(core_axis_name="c", subcore_axis_name="s"),
    compiler_params=pltpu.CompilerParams(needs_layout_passes=False),
    scratch_types=[
        pltpu.VMEM((COLS,), jnp.float32),
        pltpu.VMEM((COLS,), jnp.float32),
        pltpu.VMEM((NGROUP * 16,), jnp.float32),
        pltpu.VMEM((NGROUP * 16 + 16,), jnp.int32),
        pltpu.VMEM((K,), jnp.float32),
        pltpu.SemaphoreType.DMA,
        pltpu.SemaphoreType.DMA,
    ],
)
def _topk_sc(x_hbm, out_hbm, x_v0, x_v1, m_v, cand_v, res_v, sem0, sem1):
    wid = lax.axis_index("s") * NC + lax.axis_index("c")
    lane = lax.iota(jnp.int32, 16)

    row0 = wid * ROWS_PER_W
    cp0 = pltpu.async_copy(x_hbm.at[row0], x_v0, sem0)
    cp1 = pltpu.async_copy(x_hbm.at[row0 + 1], x_v1, sem1)

    def process_row(x_v, r):
        def bucket_body(g, T):
            base = g * 256
            acc = x_v[pl.ds(base, 16)]
            for j in range(1, 16):
                acc = jnp.maximum(acc, x_v[pl.ds(base + j * 16, 16)])
            m_v[pl.ds(g * 16, 16)] = acc
            t = acc
            T2 = []
            for s in range(8):
                T2.append(jnp.maximum(T[s], t))
                t = jnp.minimum(T[s], t)
            return tuple(T2)

        z = jnp.full((16,), NEG, jnp.float32)
        T = lax.fori_loop(0, NGROUP, bucket_body, (z,) * 8)

        AM = _neg_buf()
        for s in range(8):
            AM = _merge(AM, T[s])
        tau_v = _bcast0(AM[0])

        def comp_body(gq, offs):
            new_offs = []
            for u in range(4):
                g = u * (NGROUP // 4) + gq
                mg = m_v[pl.ds(g * 16, 16)]
                m = mg >= tau_v
                ids = g * 16 + lane
                plsc.store_compressed(
                    cand_v.at[pl.ds(u * 512 + offs[u], 16)], ids, mask=m)
                new_offs.append(
                    offs[u] + plsc.all_reduce_population_count(m)[0])
            return tuple(new_offs)

        zero = jnp.int32(0)
        counts = lax.fori_loop(0, NGROUP // 4, comp_body,
                               (zero, zero, zero, zero))

        mcount = jnp.maximum(jnp.maximum(counts[0], counts[1]),
                             jnp.maximum(counts[2], counts[3]))
        count_vs = [jnp.full((16,), c, jnp.int32) for c in counts]
        negv = jnp.full((16,), NEG, jnp.float32)

        def cand_body(i, Fs):
            i_v = jnp.full((16,), i, jnp.int32)
            new_Fs = []
            for u in range(4):
                cid = cand_v[pl.ds(u * 512 + i, 16)][0]
                idx = ((cid // 16) * 256 + lane * 16 + (cid % 16)) & (COLS - 1)
                b = plsc.load_gather(x_v, [idx])
                b = jnp.where(i_v < count_vs[u], b, negv)
                new_Fs.append(_merge(Fs[u], b))
            return tuple(new_Fs)

        Fs = lax.fori_loop(0, mcount, cand_body, (_neg_buf(),) * 4)

        def merge64(A, B):
            k = [jnp.maximum(A[i], lax.rev(B[3 - i], dimensions=(0,)))
                 for i in range(4)]
            p0 = jnp.minimum(k[0], k[2])
            p2 = jnp.maximum(k[0], k[2])
            p1 = jnp.minimum(k[1], k[3])
            p3 = jnp.maximum(k[1], k[3])
            q0 = jnp.minimum(p0, p1)
            q1 = jnp.maximum(p0, p1)
            q2 = jnp.minimum(p2, p3)
            q3 = jnp.maximum(p2, p3)
            return (_sort_asc(q0), _sort_asc(q1),
                    _sort_asc(q2), _sort_asc(q3))

        F = merge64(merge64(Fs[0], Fs[1]), merge64(Fs[2], Fs[3]))

        for j in range(4):
            res_v[pl.ds(j * 16, 16)] = lax.rev(F[3 - j], dimensions=(0,))
        pltpu.sync_copy(res_v, out_hbm.at[r])

    cp0.wait()
    process_row(x_v0, row0)
    cp1.wait()
    process_row(x_v1, row0 + 1)


def kernel(x):
    return _topk_sc(x).reshape(1, ROWS * K)
